# baseline (speedup 1.0000x reference)
import jax
import jax.numpy as jnp
from jax import lax
from jax.experimental import pallas as pl
from jax.experimental.pallas import tpu as pltpu
from jax.experimental.pallas import tpu_sc as plsc

BATCH = 1024
MAX_LENGTH = 200
BL = BATCH * MAX_LENGTH
D_OUT = 306
D_SMALL = 50
WORD = 128
LANES = 16
NC, NS = 2, 16
NW = NC * NS
PER_W = BL // NW
CHUNK = 40
ITERS = PER_W // CHUNK
NBUF = 2


def _full(val):
    return jnp.full((LANES,), val, dtype=jnp.int32)


def _sc_body(pos1_h, pos2_h, path_h, chk_h, sem_h,
             p1t_h, p2t_h, ptht_h, word_h,
             out_h,
             p1t_v, p2t_v, ptht_v,
             i1_v, i2_v, ip_v, ic_v, is_v,
             cbuf, sbuf, asml, abuf,
             idx_sem0, idx_sem1, gat_sem0, gat_sem1, out_sem):
    wid = lax.axis_index("s") * NC + lax.axis_index("c")
    idx_hs = (pos1_h, pos2_h, path_h, chk_h, sem_h)
    idx_vs = (i1_v, i2_v, ip_v, ic_v, is_v)
    idx_sem = (idx_sem0, idx_sem1)
    gat_sem = (gat_sem0, gat_sem1)

    pltpu.sync_copy(p1t_h, p1t_v)
    pltpu.sync_copy(p2t_h, p2t_v)
    pltpu.sync_copy(ptht_h, ptht_v)

    lane = lax.iota(jnp.int32, LANES)

    def fire_idx(s, t):
        base = wid * PER_W + t * CHUNK
        for h, v in zip(idx_hs, idx_vs):
            pltpu.async_copy(h.at[pl.ds(base, CHUNK)], v.at[s], idx_sem[s])

    def drain_idx(s):
        for h, v in zip(idx_hs, idx_vs):
            pltpu.make_async_copy(h.at[pl.ds(0, CHUNK)], v.at[s], idx_sem[s]).wait()

    def out_copy(base):
        b = base // MAX_LENGTH
        l0 = base % MAX_LENGTH
        return (abuf, out_h.at[b, pl.ds(l0, CHUNK)])

    fire_idx(0, 0)
    fire_idx(1, 1)

    def step(g, s):
        t = g * NBUF + s
        base = wid * PER_W + t * CHUNK
        drain_idx(s)
        g1 = pltpu.async_copy(word_h.at[ic_v.at[s]], cbuf.at[s], gat_sem[s])
        g2 = pltpu.async_copy(word_h.at[is_v.at[s]], sbuf.at[s], gat_sem[s])

        @plsc.parallel_loop(0, CHUNK // LANES, step=1)
        def rows_body(r16):
            rows = (r16 * LANES + lane) * D_SMALL
            p1i = i1_v[s, pl.ds(r16 * LANES, LANES)] * 5
            p2i = i2_v[s, pl.ds(r16 * LANES, LANES)] * 5
            ppi = ip_v[s, pl.ds(r16 * LANES, LANES)] * 40
            for f in range(5):
                v = plsc.load_gather(p1t_v, [p1i + f])
                plsc.store_scatter(asml, [rows + f], v)
            for f in range(5):
                v = plsc.load_gather(p2t_v, [p2i + f])
                plsc.store_scatter(asml, [rows + (5 + f)], v)
            for f in range(40):
                v = plsc.load_gather(ptht_v, [ppi + f])
                plsc.store_scatter(asml, [rows + (10 + f)], v)

        g1.wait()
        g2.wait()

        @pl.when(jnp.logical_or(s == 1, g >= 1))
        def _():
            pltpu.make_async_copy(*out_copy(0), out_sem).wait()

        small_plan = ((0, 0), (16, 16), (32, 32), (34, 34))
        word_plan = (
            (0, 0), (16, 16), (32, 32), (48, 48), (62, 62),
            (78, 78), (94, 94), (110, 110), (112, 112),
        )

        @plsc.parallel_loop(0, CHUNK, step=1, unroll=2)
        def copy_body(r):
            so = r * D_SMALL
            for d, w in small_plan:
                abuf[r, pl.ds(d, LANES)] = asml[pl.ds(so + w, LANES)]
            for d, w in word_plan:
                abuf[r, pl.ds(D_SMALL + d, LANES)] = cbuf[s, r, pl.ds(w, LANES)]
            for d, w in word_plan:
                abuf[r, pl.ds(D_SMALL + WORD + d, LANES)] = sbuf[s, r, pl.ds(w, LANES)]

        pltpu.async_copy(*out_copy(base), out_sem)

        @pl.when(t + NBUF < ITERS)
        def _():
            fire_idx(s, t + NBUF)

    def pair_body(g, carry):
        for s in range(NBUF):
            step(g, s)
        return carry

    lax.fori_loop(0, ITERS // NBUF, pair_body, 0, unroll=False)

    pltpu.make_async_copy(*out_copy(0), out_sem).wait()


@jax.jit
def _run(pos1, pos2, path, chunks, semantics,
         pos1_table, pos2_table, path_table, word_table):
    mesh = plsc.VectorSubcoreMesh(
        core_axis_name="c", subcore_axis_name="s",
        num_cores=NC, num_subcores=NS)
    f = pl.kernel(
        _sc_body,
        out_type=jax.ShapeDtypeStruct((BATCH, MAX_LENGTH, D_OUT), jnp.float32),
        mesh=mesh,
        scratch_types=[
            pltpu.VMEM((400 * 5,), jnp.float32),
            pltpu.VMEM((400 * 5,), jnp.float32),
            pltpu.VMEM((400 * 40,), jnp.float32),
            pltpu.VMEM((NBUF, CHUNK), jnp.int32),
            pltpu.VMEM((NBUF, CHUNK), jnp.int32),
            pltpu.VMEM((NBUF, CHUNK), jnp.int32),
            pltpu.VMEM((NBUF, CHUNK), jnp.int32),
            pltpu.VMEM((NBUF, CHUNK), jnp.int32),
            pltpu.VMEM((NBUF, CHUNK, WORD), jnp.float32),
            pltpu.VMEM((NBUF, CHUNK, WORD), jnp.float32),
            pltpu.VMEM((CHUNK * D_SMALL + LANES,), jnp.float32),
            pltpu.VMEM((CHUNK, D_OUT), jnp.float32),
            pltpu.SemaphoreType.DMA,
            pltpu.SemaphoreType.DMA,
            pltpu.SemaphoreType.DMA,
            pltpu.SemaphoreType.DMA,
            pltpu.SemaphoreType.DMA,
        ],
        compiler_params=pltpu.CompilerParams(
            use_tc_tiling_on_sc=True, needs_layout_passes=False),
        name="bert_embed_concat_sc",
    )
    return f(pos1, pos2, path, chunks, semantics,
             pos1_table, pos2_table, path_table, word_table)


def kernel(token, att_mask, pos1, pos2, path, chunks, semantics,
           pos1_table, pos2_table, path_table, word_table):
    del token, att_mask
    return _run(pos1.reshape(BL), pos2.reshape(BL), path.reshape(BL),
                chunks.reshape(BL), semantics.reshape(BL),
                pos1_table.reshape(-1), pos2_table.reshape(-1),
                path_table.reshape(-1), word_table)


# speedup vs baseline: 1.3083x; 1.3083x over previous
import jax
import jax.numpy as jnp
from jax import lax
from jax.experimental import pallas as pl
from jax.experimental.pallas import tpu as pltpu
from jax.experimental.pallas import tpu_sc as plsc

BATCH = 1024
MAX_LENGTH = 200
BL = BATCH * MAX_LENGTH
D_OUT = 306
D_SMALL = 50
WORD = 128
LANES = 16
NC, NS = 2, 16
NW = NC * NS
PER_W = BL // NW
CHUNK = 80
ITERS = PER_W // CHUNK
NBUF = 2


def _full(val):
    return jnp.full((LANES,), val, dtype=jnp.int32)


def _sc_body(pos1_h, pos2_h, path_h, chk_h, sem_h,
             p1t_h, p2t_h, ptht_h, word_h,
             out_h,
             p1t_v, p2t_v, ptht_v,
             i1_v, i2_v, ip_v, ic_v, is_v,
             cbuf, sbuf, asml, abuf,
             idx_sem0, idx_sem1, gat_sem0, gat_sem1, out_sem):
    wid = lax.axis_index("s") * NC + lax.axis_index("c")
    idx_hs = (pos1_h, pos2_h, path_h, chk_h, sem_h)
    idx_vs = (i1_v, i2_v, ip_v, ic_v, is_v)
    idx_sem = (idx_sem0, idx_sem1)
    gat_sem = (gat_sem0, gat_sem1)

    pltpu.sync_copy(p1t_h, p1t_v)
    pltpu.sync_copy(p2t_h, p2t_v)
    pltpu.sync_copy(ptht_h, ptht_v)

    lane = lax.iota(jnp.int32, LANES)

    def fire_idx(s, t):
        base = wid * PER_W + t * CHUNK
        for h, v in zip(idx_hs, idx_vs):
            pltpu.async_copy(h.at[pl.ds(base, CHUNK)], v.at[s], idx_sem[s])

    def drain_idx(s):
        for h, v in zip(idx_hs, idx_vs):
            pltpu.make_async_copy(h.at[pl.ds(0, CHUNK)], v.at[s], idx_sem[s]).wait()

    def out_copy(base):
        return (abuf, out_h.at[pl.ds(base, CHUNK)])

    fire_idx(0, 0)
    fire_idx(1, 1)

    def fire_gathers(s):
        pltpu.async_copy(word_h.at[ic_v.at[s]], cbuf.at[s], gat_sem[s])
        pltpu.async_copy(word_h.at[is_v.at[s]], sbuf.at[s], gat_sem[s])

    def drain_gathers(s):
        pltpu.make_async_copy(word_h.at[ic_v.at[s]], cbuf.at[s], gat_sem[s]).wait()
        pltpu.make_async_copy(word_h.at[is_v.at[s]], sbuf.at[s], gat_sem[s]).wait()

    def smalls(s):
        @plsc.parallel_loop(0, CHUNK // LANES, step=1)
        def rows_body(r16):
            rows = (r16 * LANES + lane) * D_SMALL
            p1i = i1_v[s, pl.ds(r16 * LANES, LANES)] * 5
            p2i = i2_v[s, pl.ds(r16 * LANES, LANES)] * 5
            ppi = ip_v[s, pl.ds(r16 * LANES, LANES)] * 40
            for f in range(5):
                v = plsc.load_gather(p1t_v, [p1i + f])
                plsc.store_scatter(asml, [rows + f], v)
            for f in range(5):
                v = plsc.load_gather(p2t_v, [p2i + f])
                plsc.store_scatter(asml, [rows + (5 + f)], v)
            for f in range(40):
                v = plsc.load_gather(ptht_v, [ppi + f])
                plsc.store_scatter(asml, [rows + (10 + f)], v)

    small_plan = ((0, 0), (16, 16), (32, 32), (34, 34))
    word_plan = (
        (0, 0), (16, 16), (32, 32), (48, 48), (62, 62),
        (78, 78), (94, 94), (110, 110), (112, 112),
    )

    def copies(s):
        @plsc.parallel_loop(0, CHUNK, step=1, unroll=2)
        def copy_body(r):
            so = r * D_SMALL
            for d, w in small_plan:
                abuf[r, pl.ds(d, LANES)] = asml[pl.ds(so + w, LANES)]
            for d, w in word_plan:
                abuf[r, pl.ds(D_SMALL + d, LANES)] = cbuf[s, r, pl.ds(w, LANES)]
            for d, w in word_plan:
                abuf[r, pl.ds(D_SMALL + WORD + d, LANES)] = sbuf[s, r, pl.ds(w, LANES)]

    drain_idx(0)
    fire_gathers(0)
    smalls(0)

    def step(g, s):
        t = g * NBUF + s
        base = wid * PER_W + t * CHUNK
        sp = 1 - s
        drain_gathers(s)

        @pl.when(t + 1 < ITERS)
        def _():
            drain_idx(sp)
            fire_gathers(sp)

        @pl.when(t >= 1)
        def _():
            pltpu.make_async_copy(*out_copy(0), out_sem).wait()

        copies(s)
        pltpu.async_copy(*out_copy(base), out_sem)

        @pl.when(t + 1 < ITERS)
        def _():
            smalls(sp)

        @pl.when(t + NBUF < ITERS)
        def _():
            fire_idx(s, t + NBUF)

    def pair_body(g, carry):
        for s in range(NBUF):
            step(g, s)
        return carry

    lax.fori_loop(0, ITERS // NBUF, pair_body, 0, unroll=False)

    pltpu.make_async_copy(*out_copy(0), out_sem).wait()


@jax.jit
def _run(pos1, pos2, path, chunks, semantics,
         pos1_table, pos2_table, path_table, word_table):
    mesh = plsc.VectorSubcoreMesh(
        core_axis_name="c", subcore_axis_name="s",
        num_cores=NC, num_subcores=NS)
    f = pl.kernel(
        _sc_body,
        out_type=jax.ShapeDtypeStruct((BL, D_OUT), jnp.float32),
        mesh=mesh,
        scratch_types=[
            pltpu.VMEM((400 * 5,), jnp.float32),
            pltpu.VMEM((400 * 5,), jnp.float32),
            pltpu.VMEM((400 * 40,), jnp.float32),
            pltpu.VMEM((NBUF, CHUNK), jnp.int32),
            pltpu.VMEM((NBUF, CHUNK), jnp.int32),
            pltpu.VMEM((NBUF, CHUNK), jnp.int32),
            pltpu.VMEM((NBUF, CHUNK), jnp.int32),
            pltpu.VMEM((NBUF, CHUNK), jnp.int32),
            pltpu.VMEM((NBUF, CHUNK, WORD), jnp.float32),
            pltpu.VMEM((NBUF, CHUNK, WORD), jnp.float32),
            pltpu.VMEM((CHUNK * D_SMALL + LANES,), jnp.float32),
            pltpu.VMEM((CHUNK, D_OUT), jnp.float32),
            pltpu.SemaphoreType.DMA,
            pltpu.SemaphoreType.DMA,
            pltpu.SemaphoreType.DMA,
            pltpu.SemaphoreType.DMA,
            pltpu.SemaphoreType.DMA,
        ],
        compiler_params=pltpu.CompilerParams(
            use_tc_tiling_on_sc=True, needs_layout_passes=False),
        name="bert_embed_concat_sc",
    )
    return f(pos1, pos2, path, chunks, semantics,
             pos1_table, pos2_table, path_table, word_table)


def kernel(token, att_mask, pos1, pos2, path, chunks, semantics,
           pos1_table, pos2_table, path_table, word_table):
    del token, att_mask
    out = _run(pos1.reshape(BL), pos2.reshape(BL), path.reshape(BL),
               chunks.reshape(BL), semantics.reshape(BL),
               pos1_table.reshape(-1), pos2_table.reshape(-1),
               path_table.reshape(-1), word_table)
    return out.reshape(BATCH, MAX_LENGTH, D_OUT)
